# Initial kernel scaffold; baseline (speedup 1.0000x reference)
import jax
import jax.numpy as jnp
from jax.experimental import pallas as pl


def kernel(images, gt, mask):
    raise NotImplementedError("write your pallas kernel here")


# speedup vs baseline: 1.1012x; 1.1012x over previous
import jax
import jax.numpy as jnp
from jax import lax
from jax.experimental import pallas as pl
from jax.experimental.pallas import tpu as pltpu

SIGMA = 5.0
B, C, H, W = 128, 3, 384, 384


def _body(mask_ref, gt_ref, img_ref, out_ref):
    b = pl.program_id(0)
    m = mask_ref[b]
    x0 = gt_ref[b, 0]
    y0 = gt_ref[b, 1]
    img = img_ref[0]
    xs = lax.broadcasted_iota(jnp.int32, (H, W), 1).astype(jnp.float32)
    ys = lax.broadcasted_iota(jnp.int32, (H, W), 0).astype(jnp.float32)
    inv = 1.0 / (2.0 * SIGMA * SIGMA)
    heat = jnp.exp(-((xs - x0) ** 2 + (ys - y0) ** 2) * inv)
    out_ref[0, 1] = img[1]
    out_ref[0, 2] = img[2]
    out_ref[0, 0] = jnp.where(m != 0, heat, img[0])


def kernel(images, gt, mask):
    mask_i = mask.astype(jnp.int32)
    return pl.pallas_call(
        _body,
        grid=(B,),
        in_specs=[
            pl.BlockSpec(memory_space=pltpu.SMEM),
            pl.BlockSpec(memory_space=pltpu.SMEM),
            pl.BlockSpec((1, C, H, W), lambda b: (b, 0, 0, 0)),
        ],
        out_specs=pl.BlockSpec((1, C, H, W), lambda b: (b, 0, 0, 0)),
        out_shape=jax.ShapeDtypeStruct((B, C, H, W), jnp.float32),
    )(mask_i, gt, images)
